# baseline (speedup 1.0000x reference)
import functools

import jax
import jax.numpy as jnp
from jax.experimental import pallas as pl
from jax.experimental.pallas import tpu as pltpu

EMB = 2048
NE = 16
TOKENS = 4 * 4096
BLK = 512


def _gating_body(x_ref, wt_ref, gwt_ref, tkwt_ref, tkit_ref):
    x = x_ref[...]
    wt = wt_ref[...]
    logits = jnp.dot(x, wt, preferred_element_type=jnp.float32)
    lg = logits.T

    m = jnp.max(lg, axis=0, keepdims=True)
    e = jnp.exp(lg - m)
    s = jnp.sum(e, axis=0, keepdims=True)
    gw = e / s
    gwt_ref[...] = gw

    row = jax.lax.broadcasted_iota(jnp.int32, gw.shape, 0)
    m1 = jnp.max(gw, axis=0, keepdims=True)
    i1 = jnp.min(jnp.where(gw == m1, row, NE), axis=0, keepdims=True)
    masked = jnp.where(row == i1, -jnp.inf, gw)
    m2 = jnp.max(masked, axis=0, keepdims=True)
    i2 = jnp.min(jnp.where(masked == m2, row, NE), axis=0, keepdims=True)

    e2 = jnp.exp(m2 - m1)
    denom = 1.0 + e2
    row2 = jax.lax.broadcasted_iota(jnp.int32, (2, gw.shape[1]), 0)
    tkwt_ref[...] = jnp.where(row2 == 0, 1.0 / denom, e2 / denom)
    tkit_ref[...] = jnp.where(row2 == 0, i1, i2)


@functools.partial(jax.jit, static_argnames=("interpret",))
def kernel(x, W, interpret=False):
    xf = x.reshape(TOKENS, EMB)
    wt = W.T
    grid = (TOKENS // BLK,)
    gwt, tkwt, tkit = pl.pallas_call(
        _gating_body,
        grid=grid,
        in_specs=[
            pl.BlockSpec((BLK, EMB), lambda i: (i, 0)),
            pl.BlockSpec((EMB, NE), lambda i: (0, 0)),
        ],
        out_specs=[
            pl.BlockSpec((NE, BLK), lambda i: (0, i)),
            pl.BlockSpec((2, BLK), lambda i: (0, i)),
            pl.BlockSpec((2, BLK), lambda i: (0, i)),
        ],
        out_shape=[
            jax.ShapeDtypeStruct((NE, TOKENS), jnp.float32),
            jax.ShapeDtypeStruct((2, TOKENS), jnp.float32),
            jax.ShapeDtypeStruct((2, TOKENS), jnp.int32),
        ],
        interpret=interpret,
        compiler_params=pltpu.CompilerParams(
            dimension_semantics=("arbitrary",),
        ),
    )(xf, wt)
    B, S = x.shape[0], x.shape[1]
    return (gwt.T.reshape(B, S, NE), tkwt.T.reshape(B, S, 2),
            tkit.T.reshape(B, S, 2))


# speedup vs baseline: 1.1072x; 1.1072x over previous
import functools

import jax
import jax.numpy as jnp
from jax.experimental import pallas as pl
from jax.experimental.pallas import tpu as pltpu

EMB = 2048
NE = 16
TOKENS = 4 * 4096
BLK = 1024
NBLK = TOKENS // BLK
NBUF = 4


def _gating_body(x_hbm, wt_ref, gwt_ref, tkwt_ref, tkit_ref, xbuf, sems):
    i = pl.program_id(0)

    @pl.when(i == 0)
    def _prolog():
        for b in range(NBUF - 1):
            pltpu.make_async_copy(
                x_hbm.at[pl.ds(b * BLK, BLK), :], xbuf.at[b], sems.at[b]
            ).start()

    nxt = i + NBUF - 1

    @pl.when(nxt < NBLK)
    def _prefetch():
        slot = jax.lax.rem(nxt, NBUF)
        pltpu.make_async_copy(
            x_hbm.at[pl.ds(nxt * BLK, BLK), :], xbuf.at[slot], sems.at[slot]
        ).start()

    cur = jax.lax.rem(i, NBUF)
    pltpu.make_async_copy(
        x_hbm.at[pl.ds(i * BLK, BLK), :], xbuf.at[cur], sems.at[cur]
    ).wait()

    x = xbuf[cur]
    wt = wt_ref[...]
    logits = jnp.dot(x, wt, preferred_element_type=jnp.float32)
    lg = logits.T

    m = jnp.max(lg, axis=0, keepdims=True)
    e = jnp.exp(lg - m)
    s = jnp.sum(e, axis=0, keepdims=True)
    gw = e / s
    gwt_ref[...] = gw

    row = jax.lax.broadcasted_iota(jnp.int32, gw.shape, 0)
    m1 = jnp.max(gw, axis=0, keepdims=True)
    i1 = jnp.min(jnp.where(gw == m1, row, NE), axis=0, keepdims=True)
    masked = jnp.where(row == i1, -jnp.inf, gw)
    m2 = jnp.max(masked, axis=0, keepdims=True)
    i2 = jnp.min(jnp.where(masked == m2, row, NE), axis=0, keepdims=True)

    e2 = jnp.exp(m2 - m1)
    denom = 1.0 + e2
    row2 = jax.lax.broadcasted_iota(jnp.int32, (2, gw.shape[1]), 0)
    tkwt_ref[...] = jnp.where(row2 == 0, 1.0 / denom, e2 / denom)
    tkit_ref[...] = jnp.where(row2 == 0, i1, i2)


@functools.partial(jax.jit, static_argnames=("interpret",))
def kernel(x, W, interpret=False):
    xf = x.reshape(TOKENS, EMB)
    wt = W.T
    gwt, tkwt, tkit = pl.pallas_call(
        _gating_body,
        grid=(NBLK,),
        in_specs=[
            pl.BlockSpec(memory_space=pltpu.MemorySpace.HBM),
            pl.BlockSpec((EMB, NE), lambda i: (0, 0)),
        ],
        out_specs=[
            pl.BlockSpec((NE, BLK), lambda i: (0, i)),
            pl.BlockSpec((2, BLK), lambda i: (0, i)),
            pl.BlockSpec((2, BLK), lambda i: (0, i)),
        ],
        out_shape=[
            jax.ShapeDtypeStruct((NE, TOKENS), jnp.float32),
            jax.ShapeDtypeStruct((2, TOKENS), jnp.float32),
            jax.ShapeDtypeStruct((2, TOKENS), jnp.int32),
        ],
        scratch_shapes=[
            pltpu.MemorySpace.VMEM((NBUF, BLK, EMB), jnp.float32),
            pltpu.SemaphoreType.DMA((NBUF,)),
        ],
        interpret=interpret,
        compiler_params=pltpu.CompilerParams(
            dimension_semantics=("arbitrary",),
        ),
    )(xf, wt)
    B, S = x.shape[0], x.shape[1]
    return (gwt.T.reshape(B, S, NE), tkwt.T.reshape(B, S, 2),
            tkit.T.reshape(B, S, 2))


# speedup vs baseline: 1.1628x; 1.0502x over previous
import functools

import jax
import jax.numpy as jnp
from jax.experimental import pallas as pl
from jax.experimental.pallas import tpu as pltpu

EMB = 2048
NE = 16
TOKENS = 4 * 4096
BLK = 1024


def _gating_body(x_ref, wt_ref, gwt_ref, tkwt_ref, tkit_ref):
    x = x_ref[...]
    wt = wt_ref[...]
    logits = jnp.dot(x, wt, preferred_element_type=jnp.float32)
    lg = logits.T

    m = jnp.max(lg, axis=0, keepdims=True)
    e = jnp.exp(lg - m)
    s = jnp.sum(e, axis=0, keepdims=True)
    gw = e / s
    gwt_ref[...] = gw

    row = jax.lax.broadcasted_iota(jnp.int32, gw.shape, 0)
    m1 = jnp.max(gw, axis=0, keepdims=True)
    i1 = jnp.min(jnp.where(gw == m1, row, NE), axis=0, keepdims=True)
    masked = jnp.where(row == i1, -jnp.inf, gw)
    m2 = jnp.max(masked, axis=0, keepdims=True)
    i2 = jnp.min(jnp.where(masked == m2, row, NE), axis=0, keepdims=True)

    e2 = jnp.exp(m2 - m1)
    denom = 1.0 + e2
    row2 = jax.lax.broadcasted_iota(jnp.int32, (2, gw.shape[1]), 0)
    tkwt_ref[...] = jnp.where(row2 == 0, 1.0 / denom, e2 / denom)
    tkit_ref[...] = jnp.where(row2 == 0, i1, i2)


@functools.partial(jax.jit, static_argnames=("interpret",))
def kernel(x, W, interpret=False):
    xf = x.reshape(TOKENS, EMB)
    wt = W.T
    grid = (TOKENS // BLK,)
    gwt, tkwt, tkit = pl.pallas_call(
        _gating_body,
        grid=grid,
        in_specs=[
            pl.BlockSpec((BLK, EMB), lambda i: (i, 0)),
            pl.BlockSpec((EMB, NE), lambda i: (0, 0)),
        ],
        out_specs=[
            pl.BlockSpec((NE, BLK), lambda i: (0, i)),
            pl.BlockSpec((2, BLK), lambda i: (0, i)),
            pl.BlockSpec((2, BLK), lambda i: (0, i)),
        ],
        out_shape=[
            jax.ShapeDtypeStruct((NE, TOKENS), jnp.float32),
            jax.ShapeDtypeStruct((2, TOKENS), jnp.float32),
            jax.ShapeDtypeStruct((2, TOKENS), jnp.int32),
        ],
        interpret=interpret,
        compiler_params=pltpu.CompilerParams(
            dimension_semantics=("arbitrary",),
        ),
    )(xf, wt)
    B, S = x.shape[0], x.shape[1]
    return (gwt.T.reshape(B, S, NE), tkwt.T.reshape(B, S, 2),
            tkit.T.reshape(B, S, 2))
